# baseline (speedup 1.0000x reference)
import jax
import jax.numpy as jnp
from jax.experimental import pallas as pl
from jax.experimental.pallas import tpu as pltpu

_M = 1048576
_C = 16384
_GRID = _M // _C


def _body(x_hbm, a_ref, index_ref, o_ref, buf, sem):
    b = pl.program_id(0)

    @pl.when(b == 0)
    def _():
        pltpu.make_async_copy(
            x_hbm.at[:, pl.ds(0, _C)], buf.at[0], sem.at[0]).start()

    @pl.when(b + 1 < _GRID)
    def _():
        nxt = (b + 1) % 2
        pltpu.make_async_copy(
            x_hbm.at[:, pl.ds((b + 1) * _C, _C)], buf.at[nxt], sem.at[nxt]).start()

    slot = b % 2
    pltpu.make_async_copy(
        x_hbm.at[:, pl.ds(b * _C, _C)], buf.at[slot], sem.at[slot]).wait()
    o_ref[...] = buf[slot]

    col_lo = b * _C
    for k in range(3):
        idx = index_ref[k]
        rel = idx - col_lo
        in_blk = jnp.logical_and(idx >= col_lo, idx < col_lo + _C)

        @pl.when(in_blk)
        def _():
            win = pl.multiple_of((rel // 128) * 128, 128)
            lane = rel - (rel // 128) * 128
            lanes = jax.lax.broadcasted_iota(jnp.int32, (1, 128), 1)
            hit = lanes == lane
            for j in range(3):
                sub = o_ref[j:j + 1, pl.ds(win, 128)]
                upd = jnp.where(hit, a_ref[k, j], 0.0)
                o_ref[j:j + 1, pl.ds(win, 128)] = sub + upd


def kernel(x, a, index):
    out = pl.pallas_call(
        _body,
        grid=(_GRID,),
        in_specs=[
            pl.BlockSpec(memory_space=pl.ANY),
            pl.BlockSpec(memory_space=pltpu.SMEM),
            pl.BlockSpec(memory_space=pltpu.SMEM),
        ],
        out_specs=pl.BlockSpec((3, _C), lambda i: (0, i)),
        out_shape=jax.ShapeDtypeStruct((3, _M), jnp.float32),
        scratch_shapes=[
            pltpu.VMEM((2, 3, _C), jnp.float32),
            pltpu.SemaphoreType.DMA((2,)),
        ],
    )(x, a, index.astype(jnp.int32))
    return jnp.transpose(out, (1, 0))


# speedup vs baseline: 2.0751x; 2.0751x over previous
import jax
import jax.numpy as jnp
from jax.experimental import pallas as pl
from jax.experimental.pallas import tpu as pltpu

_M = 1048576
_C = 32768
_GRID = _M // _C


def _body(x_ref, a_ref, index_ref, o_ref):
    b = pl.program_id(0)
    o_ref[...] = x_ref[:, pl.ds(b * _C, _C)]

    col_lo = b * _C
    for k in range(3):
        idx = index_ref[k]
        rel = idx - col_lo
        in_blk = jnp.logical_and(idx >= col_lo, idx < col_lo + _C)

        @pl.when(in_blk)
        def _():
            win = pl.multiple_of((rel // 128) * 128, 128)
            lane = rel - (rel // 128) * 128
            lanes = jax.lax.broadcasted_iota(jnp.int32, (1, 128), 1)
            hit = lanes == lane
            for j in range(3):
                sub = o_ref[j:j + 1, pl.ds(win, 128)]
                upd = jnp.where(hit, a_ref[k, j], 0.0)
                o_ref[j:j + 1, pl.ds(win, 128)] = sub + upd


def kernel(x, a, index):
    out = pl.pallas_call(
        _body,
        grid=(_GRID,),
        in_specs=[
            pl.BlockSpec(memory_space=pltpu.VMEM),
            pl.BlockSpec(memory_space=pltpu.SMEM),
            pl.BlockSpec(memory_space=pltpu.SMEM),
        ],
        out_specs=pl.BlockSpec((3, _C), lambda i: (0, i)),
        out_shape=jax.ShapeDtypeStruct((3, _M), jnp.float32),
    )(x, a, index.astype(jnp.int32))
    return jnp.transpose(out, (1, 0))


# speedup vs baseline: 3.4338x; 1.6548x over previous
import jax
import jax.numpy as jnp
from jax.experimental import pallas as pl
from jax.experimental.pallas import tpu as pltpu

_M = 1048576
_K = 8
_CH = _M // _K


def _win(index_ref, k):
    return pl.multiple_of((index_ref[k] // 128) * 128, 128)


def _body(x_ref, a_ref, index_ref, o_hbm, tbuf, csem, wsem):
    for c in range(_K):
        pltpu.make_async_copy(
            x_ref.at[:, pl.ds(c * _CH, _CH)],
            o_hbm.at[:, pl.ds(c * _CH, _CH)],
            csem.at[c]).start()

    lanes = jax.lax.broadcasted_iota(jnp.int32, (1, 128), 1)
    for k in range(3):
        win = _win(index_ref, k)
        rows = []
        for j in range(3):
            r = x_ref[j:j + 1, pl.ds(win, 128)]
            for k2 in range(3):
                rel = index_ref[k2] - win
                r = r + jnp.where(lanes == rel, a_ref[k2, j], 0.0)
            rows.append(r)
        tbuf[k] = jnp.concatenate(rows, axis=0)

    for c in range(_K):
        pltpu.make_async_copy(
            x_ref.at[:, pl.ds(c * _CH, _CH)],
            o_hbm.at[:, pl.ds(c * _CH, _CH)],
            csem.at[c]).wait()

    for k in range(3):
        pltpu.make_async_copy(
            tbuf.at[k],
            o_hbm.at[:, pl.ds(_win(index_ref, k), 128)],
            wsem.at[k]).start()
    for k in range(3):
        pltpu.make_async_copy(
            tbuf.at[k],
            o_hbm.at[:, pl.ds(_win(index_ref, k), 128)],
            wsem.at[k]).wait()


def kernel(x, a, index):
    out = pl.pallas_call(
        _body,
        in_specs=[
            pl.BlockSpec(memory_space=pltpu.VMEM),
            pl.BlockSpec(memory_space=pltpu.SMEM),
            pl.BlockSpec(memory_space=pltpu.SMEM),
        ],
        out_specs=pl.BlockSpec(memory_space=pl.ANY),
        out_shape=jax.ShapeDtypeStruct((3, _M), jnp.float32),
        scratch_shapes=[
            pltpu.VMEM((3, 3, 128), jnp.float32),
            pltpu.SemaphoreType.DMA((_K,)),
            pltpu.SemaphoreType.DMA((3,)),
        ],
    )(x, a, index.astype(jnp.int32))
    return jnp.transpose(out, (1, 0))


# speedup vs baseline: 3.5868x; 1.0446x over previous
import jax
import jax.numpy as jnp
from jax.experimental import pallas as pl
from jax.experimental.pallas import tpu as pltpu

_M = 1048576
_K = 8
_CH = _M // _K


def _win(index_ref, k):
    return pl.multiple_of((index_ref[k] // 128) * 128, 128)


def _body(x_ref, a_ref, index_ref, o_hbm, tbuf, csem, wsem):
    for c in range(_K):
        pltpu.make_async_copy(
            x_ref.at[:, pl.ds(c * _CH, _CH)],
            o_hbm.at[:, pl.ds(c * _CH, _CH)],
            csem.at[c]).start()

    lanes = jax.lax.broadcasted_iota(jnp.int32, (1, 128), 1)
    for k in range(3):
        win = _win(index_ref, k)
        rows = []
        for j in range(3):
            r = x_ref[j:j + 1, pl.ds(win, 128)]
            for k2 in range(3):
                rel = index_ref[k2] - win
                r = r + jnp.where(lanes == rel, a_ref[k2, j], 0.0)
            rows.append(r)
        tbuf[k] = jnp.concatenate(rows, axis=0)

    for c in range(_K):
        pltpu.make_async_copy(
            x_ref.at[:, pl.ds(c * _CH, _CH)],
            o_hbm.at[:, pl.ds(c * _CH, _CH)],
            csem.at[c]).wait()
        for k in range(3):
            win = _win(index_ref, k)

            @pl.when(win // _CH == c)
            def _():
                pltpu.make_async_copy(
                    tbuf.at[k],
                    o_hbm.at[:, pl.ds(win, 128)],
                    wsem.at[k]).start()

    for k in range(3):
        pltpu.make_async_copy(
            tbuf.at[k],
            o_hbm.at[:, pl.ds(_win(index_ref, k), 128)],
            wsem.at[k]).wait()


def kernel(x, a, index):
    out = pl.pallas_call(
        _body,
        in_specs=[
            pl.BlockSpec(memory_space=pltpu.VMEM),
            pl.BlockSpec(memory_space=pltpu.SMEM),
            pl.BlockSpec(memory_space=pltpu.SMEM),
        ],
        out_specs=pl.BlockSpec(memory_space=pl.ANY),
        out_shape=jax.ShapeDtypeStruct((3, _M), jnp.float32),
        scratch_shapes=[
            pltpu.VMEM((3, 3, 128), jnp.float32),
            pltpu.SemaphoreType.DMA((_K,)),
            pltpu.SemaphoreType.DMA((3,)),
        ],
    )(x, a, index.astype(jnp.int32))
    return jnp.transpose(out, (1, 0))
